# baseline (speedup 1.0000x reference)
import functools

import jax
import jax.numpy as jnp
from jax.experimental import pallas as pl
from jax.experimental.pallas import tpu as pltpu

S = 512
H = 256
G = 4 * H


def _lstm_layer_body(x_ref, wihT_f, whhT_f, bias_f, wihT_b, whhT_b, bias_b,
                     out_ref, xp_f, xp_b):
    xb = x_ref[:].astype(jnp.bfloat16)
    xp_f[:] = jnp.dot(xb, wihT_f[:], preferred_element_type=jnp.float32) + bias_f[:]
    xp_b[:] = jnp.dot(xb, wihT_b[:], preferred_element_type=jnp.float32) + bias_b[:]

    def sig(x):
        return 0.5 * jnp.tanh(0.5 * x) + 0.5

    def gates(g, c):
        i = sig(g[:, 0:H])
        f = sig(g[:, H:2 * H])
        gg = jnp.tanh(g[:, 2 * H:3 * H])
        o = sig(g[:, 3 * H:4 * H])
        c2 = f * c + i * gg
        h2 = o * jnp.tanh(c2)
        return h2, c2

    def rec(h, xrow, whhT):
        hb = h.astype(jnp.bfloat16)
        parts = [jnp.dot(hb[:, c:c + 64], whhT[c:c + 64, :],
                         preferred_element_type=jnp.float32)
                 for c in range(0, 256, 64)]
        return xrow + (parts[0] + parts[1]) + (parts[2] + parts[3])

    def step(t, carry):
        h_f, c_f, h_b, c_b = carry
        tb = S - 1 - t
        g_f = rec(h_f, xp_f[pl.ds(t, 1), :], whhT_f)
        g_b = rec(h_b, xp_b[pl.ds(tb, 1), :], whhT_b)
        h_f, c_f = gates(g_f, c_f)
        h_b, c_b = gates(g_b, c_b)
        out_ref[pl.ds(t, 1), 0:H] = h_f
        out_ref[pl.ds(tb, 1), H:2 * H] = h_b
        return (h_f, c_f, h_b, c_b)

    z = jnp.zeros((1, H), jnp.float32)
    jax.lax.fori_loop(0, S, step, (z, z, z, z), unroll=2)


def _lstm_layer(x, pf, pb):
    din = x.shape[1]
    dpad = -(-din // 128) * 128
    if dpad != din:
        x = jnp.pad(x, ((0, 0), (0, dpad - din)))
    args = [x]
    for p in (pf, pb):
        wihT = jnp.transpose(p["Wih"])
        if dpad != din:
            wihT = jnp.pad(wihT, ((0, dpad - din), (0, 0)))
        args.append(wihT.astype(jnp.bfloat16))
        args.append(jnp.transpose(p["Whh"]).astype(jnp.bfloat16))
        args.append((p["bih"] + p["bhh"]).reshape(1, G))
    return pl.pallas_call(
        _lstm_layer_body,
        out_shape=jax.ShapeDtypeStruct((S, 2 * H), jnp.float32),
        scratch_shapes=[pltpu.VMEM((S, G), jnp.float32),
                        pltpu.VMEM((S, G), jnp.float32)],
    )(*args)


def _ab_body(emb_ref, embT_ref, w1aT, w1b, b1, a_out, bT_out):
    a_out[:] = jnp.dot(emb_ref[:].astype(jnp.bfloat16), w1aT[:],
                       preferred_element_type=jnp.float32) + b1[:]
    bT_out[:] = jnp.dot(w1b[:], embT_ref[:].astype(jnp.bfloat16),
                        preferred_element_type=jnp.float32)


_BI = 8


def _mlp_body(a_ref, bT_ref, w2, b2c, w3r, b3, sm_ref, logn_ref, nacc_ref):
    p = pl.program_id(0)
    np_ = pl.num_programs(0)
    jlane = jax.lax.broadcasted_iota(jnp.int32, (1, S), 1)

    @pl.when(p == 0)
    def _():
        nacc_ref[:] = jnp.zeros((1, S), jnp.float32)

    aT = a_ref[:].T
    bTv = bT_ref[:]
    h2s = []
    for k in range(_BI):
        acol = aT[:, k:k + 1]
        h1 = jnp.maximum(bTv + acol, 0.0)
        h2 = jnp.maximum(
            jnp.dot(w2[:], h1.astype(jnp.bfloat16),
                    preferred_element_type=jnp.float32) + b2c[:], 0.0)
        h2s.append(h2.astype(jnp.bfloat16))
    h2cat = jnp.concatenate(h2s, axis=1)
    rows = jnp.maximum(
        jnp.dot(w3r[:], h2cat, preferred_element_type=jnp.float32) + b3[:], 0.0)
    block = jnp.concatenate(
        [rows[:, k * S:(k + 1) * S] for k in range(_BI)], axis=0)
    ii = p * _BI + jax.lax.broadcasted_iota(jnp.int32, (_BI, S), 0)
    jj = jax.lax.broadcasted_iota(jnp.int32, (_BI, S), 1)
    offdiag = jj != ii
    blockm = jnp.where(offdiag & (jj >= 1), block, 0.0)
    sm_ref[:] = blockm
    nacc_ref[:] += jnp.sum(jnp.where(offdiag, jnp.exp(blockm), 0.0),
                           axis=0, keepdims=True)

    @pl.when(p == np_ - 1)
    def _():
        logn_ref[:] = jnp.log(nacc_ref[:])


def _pairwise_scores(emb, mlp_params):
    d = 2 * H
    w1 = mlp_params["W1"]
    a, bT = pl.pallas_call(
        _ab_body,
        out_shape=[jax.ShapeDtypeStruct((S, H), jnp.float32),
                   jax.ShapeDtypeStruct((H, S), jnp.float32)],
    )(emb, jnp.transpose(emb), jnp.transpose(w1[:, :d]).astype(jnp.bfloat16),
      w1[:, d:].astype(jnp.bfloat16), mlp_params["b1"].reshape(1, H))

    nprog = S // _BI
    sm, logn = pl.pallas_call(
        _mlp_body,
        grid=(nprog,),
        in_specs=[
            pl.BlockSpec((_BI, H), lambda p: (p, 0)),
            pl.BlockSpec((H, S), lambda p: (0, 0)),
            pl.BlockSpec((128, H), lambda p: (0, 0)),
            pl.BlockSpec((128, 1), lambda p: (0, 0)),
            pl.BlockSpec((1, 128), lambda p: (0, 0)),
            pl.BlockSpec((1, 1), lambda p: (0, 0)),
        ],
        out_specs=[
            pl.BlockSpec((_BI, S), lambda p: (p, 0)),
            pl.BlockSpec((1, S), lambda p: (0, 0)),
        ],
        out_shape=[jax.ShapeDtypeStruct((S, S), jnp.float32),
                   jax.ShapeDtypeStruct((1, S), jnp.float32)],
        scratch_shapes=[pltpu.VMEM((1, S), jnp.float32)],
    )(a, bT, mlp_params["W2"].astype(jnp.bfloat16),
      mlp_params["b2"].reshape(128, 1), mlp_params["W3"].astype(jnp.bfloat16),
      mlp_params["b3"].reshape(1, 1))
    return sm, logn[0, :]


def kernel(sentence_embedding, real_dependency_tree, lstm_params, mlp_params):
    emb = _lstm_layer(sentence_embedding, lstm_params["l0_f"], lstm_params["l0_b"])
    emb = _lstm_layer(emb, lstm_params["l1_f"], lstm_params["l1_b"])
    sm, logn = _pairwise_scores(emb, mlp_params)

    v1 = real_dependency_tree[1:, 0]
    v2 = real_dependency_tree[1:, 1]
    loss = jnp.mean(logn[v2] - sm[v1, v2])
    return loss, sm


# speedup vs baseline: 1.1895x; 1.1895x over previous
import functools

import jax
import jax.numpy as jnp
from jax import lax
from jax.experimental import pallas as pl
from jax.experimental.pallas import tpu as pltpu
from jax.experimental.pallas import tpu_sc as plsc

S = 512
H = 256
G = 4 * H


def _lstm_layer_body(x_ref, wihT_f, whhT_f, bias_f, wihT_b, whhT_b, bias_b,
                     out_ref, xp_f, xp_b):
    xb = x_ref[:].astype(jnp.bfloat16)
    xp_f[:] = jnp.dot(xb, wihT_f[:], preferred_element_type=jnp.float32) + bias_f[:]
    xp_b[:] = jnp.dot(xb, wihT_b[:], preferred_element_type=jnp.float32) + bias_b[:]

    def sig(x):
        return 0.5 * jnp.tanh(0.5 * x) + 0.5

    def gates(g, c):
        i = sig(g[:, 0:H])
        f = sig(g[:, H:2 * H])
        gg = jnp.tanh(g[:, 2 * H:3 * H])
        o = sig(g[:, 3 * H:4 * H])
        c2 = f * c + i * gg
        h2 = o * jnp.tanh(c2)
        return h2, c2

    def rec(h, xrow, whhT):
        hb = h.astype(jnp.bfloat16)
        return (xrow
                + jnp.dot(hb[:, 0:128], whhT[0:128, :],
                          preferred_element_type=jnp.float32)
                + jnp.dot(hb[:, 128:256], whhT[128:256, :],
                          preferred_element_type=jnp.float32))

    def step(t, carry):
        h_f, c_f, h_b, c_b = carry
        tb = S - 1 - t
        g_f = rec(h_f, xp_f[pl.ds(t, 1), :], whhT_f)
        g_b = rec(h_b, xp_b[pl.ds(tb, 1), :], whhT_b)
        h_f, c_f = gates(g_f, c_f)
        h_b, c_b = gates(g_b, c_b)
        out_ref[pl.ds(t, 1), 0:H] = h_f
        out_ref[pl.ds(tb, 1), H:2 * H] = h_b
        return (h_f, c_f, h_b, c_b)

    z = jnp.zeros((1, H), jnp.float32)
    jax.lax.fori_loop(0, S, step, (z, z, z, z), unroll=2)


def _lstm_layer(x, pf, pb):
    din = x.shape[1]
    dpad = -(-din // 128) * 128
    if dpad != din:
        x = jnp.pad(x, ((0, 0), (0, dpad - din)))
    args = [x]
    for p in (pf, pb):
        wihT = jnp.transpose(p["Wih"])
        if dpad != din:
            wihT = jnp.pad(wihT, ((0, dpad - din), (0, 0)))
        args.append(wihT.astype(jnp.bfloat16))
        args.append(jnp.transpose(p["Whh"]).astype(jnp.bfloat16))
        args.append((p["bih"] + p["bhh"]).reshape(1, G))
    return pl.pallas_call(
        _lstm_layer_body,
        out_shape=jax.ShapeDtypeStruct((S, 2 * H), jnp.float32),
        scratch_shapes=[pltpu.VMEM((S, G), jnp.float32),
                        pltpu.VMEM((S, G), jnp.float32)],
    )(*args)


def _ab_body(emb_ref, embT_ref, w1aT, w1b, b1, a_out, bT_out):
    a_out[:] = jnp.dot(emb_ref[:].astype(jnp.bfloat16), w1aT[:],
                       preferred_element_type=jnp.float32) + b1[:]
    bT_out[:] = jnp.dot(w1b[:], embT_ref[:].astype(jnp.bfloat16),
                        preferred_element_type=jnp.float32)


_BI = 8


def _mlp_body(a_ref, bT_ref, w2, b2c, w3r, b3, sm_ref, logn_ref, nacc_ref):
    p = pl.program_id(0)
    np_ = pl.num_programs(0)
    jlane = jax.lax.broadcasted_iota(jnp.int32, (1, S), 1)

    @pl.when(p == 0)
    def _():
        nacc_ref[:] = jnp.zeros((1, S), jnp.float32)

    aT = a_ref[:].T
    bTv = bT_ref[:]
    h2s = []
    for k in range(_BI):
        acol = aT[:, k:k + 1]
        h1 = jnp.maximum(bTv + acol, 0.0)
        h2 = jnp.maximum(
            jnp.dot(w2[:], h1.astype(jnp.bfloat16),
                    preferred_element_type=jnp.float32) + b2c[:], 0.0)
        h2s.append(h2.astype(jnp.bfloat16))
    h2cat = jnp.concatenate(h2s, axis=1)
    rows = jnp.maximum(
        jnp.dot(w3r[:], h2cat, preferred_element_type=jnp.float32) + b3[:], 0.0)
    block = jnp.concatenate(
        [rows[:, k * S:(k + 1) * S] for k in range(_BI)], axis=0)
    ii = p * _BI + jax.lax.broadcasted_iota(jnp.int32, (_BI, S), 0)
    jj = jax.lax.broadcasted_iota(jnp.int32, (_BI, S), 1)
    offdiag = jj != ii
    blockm = jnp.where(offdiag & (jj >= 1), block, 0.0)
    sm_ref[:] = blockm
    nacc_ref[:] += jnp.sum(jnp.where(offdiag, jnp.exp(blockm), 0.0),
                           axis=0, keepdims=True)

    @pl.when(p == np_ - 1)
    def _():
        logn_ref[:] = jnp.log(nacc_ref[:])


def _pairwise_scores(emb, mlp_params):
    d = 2 * H
    w1 = mlp_params["W1"]
    a, bT = pl.pallas_call(
        _ab_body,
        out_shape=[jax.ShapeDtypeStruct((S, H), jnp.float32),
                   jax.ShapeDtypeStruct((H, S), jnp.float32)],
    )(emb, jnp.transpose(emb), jnp.transpose(w1[:, :d]).astype(jnp.bfloat16),
      w1[:, d:].astype(jnp.bfloat16), mlp_params["b1"].reshape(1, H))

    nprog = S // _BI
    sm, logn = pl.pallas_call(
        _mlp_body,
        grid=(nprog,),
        in_specs=[
            pl.BlockSpec((_BI, H), lambda p: (p, 0)),
            pl.BlockSpec((H, S), lambda p: (0, 0)),
            pl.BlockSpec((128, H), lambda p: (0, 0)),
            pl.BlockSpec((128, 1), lambda p: (0, 0)),
            pl.BlockSpec((1, 128), lambda p: (0, 0)),
            pl.BlockSpec((1, 1), lambda p: (0, 0)),
        ],
        out_specs=[
            pl.BlockSpec((_BI, S), lambda p: (p, 0)),
            pl.BlockSpec((1, S), lambda p: (0, 0)),
        ],
        out_shape=[jax.ShapeDtypeStruct((S, S), jnp.float32),
                   jax.ShapeDtypeStruct((1, S), jnp.float32)],
        scratch_shapes=[pltpu.VMEM((1, S), jnp.float32)],
    )(a, bT, mlp_params["W2"].astype(jnp.bfloat16),
      mlp_params["b2"].reshape(128, 1), mlp_params["W3"].astype(jnp.bfloat16),
      mlp_params["b3"].reshape(1, 1))
    return sm, logn[0, :]


def _loss_gather_sc(smflat, logn, idx1, idx2, w):
    info = plsc.get_sparse_core_info()
    nc, ns, nl = info.num_cores, info.num_subcores, info.num_lanes
    nw = nc * ns

    @functools.partial(
        pl.kernel,
        mesh=plsc.VectorSubcoreMesh(core_axis_name="c", subcore_axis_name="s"),
        out_type=jax.ShapeDtypeStruct((nw, nl), jnp.float32),
        scratch_types=[
            pltpu.VMEM((nl,), jnp.int32),
            pltpu.VMEM((nl,), jnp.int32),
            pltpu.VMEM((nl,), jnp.float32),
            pltpu.VMEM((nl,), jnp.float32),
            pltpu.VMEM((nl,), jnp.float32),
            pltpu.VMEM((nl,), jnp.float32),
            pltpu.SemaphoreType.DMA,
        ],
    )
    def body(smflat_hbm, logn_hbm, idx1_hbm, idx2_hbm, w_hbm, out_hbm,
             idx1_v, idx2_v, w_v, smv_v, lnv_v, res_v, sem):
        wid = lax.axis_index("s") * nc + lax.axis_index("c")
        base = wid * nl
        pltpu.sync_copy(idx1_hbm.at[pl.ds(base, nl)], idx1_v)
        pltpu.sync_copy(idx2_hbm.at[pl.ds(base, nl)], idx2_v)
        pltpu.sync_copy(w_hbm.at[pl.ds(base, nl)], w_v)
        pltpu.async_copy(smflat_hbm.at[idx1_v], smv_v, sem).wait()
        pltpu.async_copy(logn_hbm.at[idx2_v], lnv_v, sem).wait()
        res_v[...] = (lnv_v[...] - smv_v[...]) * w_v[...]
        pltpu.sync_copy(res_v, out_hbm.at[wid])

    return body(smflat, logn, idx1, idx2, w)


def kernel(sentence_embedding, real_dependency_tree, lstm_params, mlp_params):
    emb = _lstm_layer(sentence_embedding, lstm_params["l0_f"], lstm_params["l0_b"])
    emb = _lstm_layer(emb, lstm_params["l1_f"], lstm_params["l1_b"])
    sm, logn = _pairwise_scores(emb, mlp_params)

    tree = real_dependency_tree.astype(jnp.int32)
    v1p = jnp.concatenate([tree[1:, 0], jnp.zeros((1,), jnp.int32)])
    v2p = jnp.concatenate([tree[1:, 1], jnp.zeros((1,), jnp.int32)])
    w = jnp.where(jnp.arange(S) < S - 1, 1.0 / (S - 1), 0.0).astype(jnp.float32)
    terms = _loss_gather_sc(sm.reshape(S * S), logn, v1p * S + v2p, v2p, w)
    loss = jnp.sum(terms)
    return loss, sm


# speedup vs baseline: 1.3372x; 1.1242x over previous
import functools

import jax
import jax.numpy as jnp
from jax import lax
from jax.experimental import pallas as pl
from jax.experimental.pallas import tpu as pltpu
from jax.experimental.pallas import tpu_sc as plsc

S = 512
H = 256
G = 4 * H


def _lstm_layer_body(x_ref, wihT_f, whhT_f, bias_f, wihT_b, whhT_b, bias_b,
                     out_ref, xp_f, xp_b):
    xb = x_ref[:].astype(jnp.bfloat16)
    xp_f[:] = jnp.dot(xb, wihT_f[:], preferred_element_type=jnp.float32) + bias_f[:]
    xp_b[:] = jnp.dot(xb, wihT_b[:], preferred_element_type=jnp.float32) + bias_b[:]

    def sig(x):
        return 0.5 * jnp.tanh(0.5 * x) + 0.5

    def gates(g, c):
        i = sig(g[:, 0:H])
        f = sig(g[:, H:2 * H])
        gg = jnp.tanh(g[:, 2 * H:3 * H])
        o = sig(g[:, 3 * H:4 * H])
        c2 = f * c + i * gg
        h2 = o * jnp.tanh(c2)
        return h2, c2

    def rec(h, xrow, whhT):
        hb = h.astype(jnp.bfloat16)
        return (xrow
                + jnp.dot(hb[:, 0:128], whhT[0:128, :],
                          preferred_element_type=jnp.float32)
                + jnp.dot(hb[:, 128:256], whhT[128:256, :],
                          preferred_element_type=jnp.float32))

    def step8(blk, carry):
        h_f, c_f, h_b, c_b = carry
        t0 = blk * 8
        tb0 = S - 8 - t0
        xf = xp_f[pl.ds(t0, 8), :]
        xb2 = xp_b[pl.ds(tb0, 8), :]
        hf_rows, hb_rows = [], []
        for k in range(8):
            g_f = rec(h_f, xf[k:k + 1, :], whhT_f)
            g_b = rec(h_b, xb2[7 - k:8 - k, :], whhT_b)
            h_f, c_f = gates(g_f, c_f)
            h_b, c_b = gates(g_b, c_b)
            hf_rows.append(h_f)
            hb_rows.append(h_b)
        out_ref[pl.ds(t0, 8), 0:H] = jnp.concatenate(hf_rows, axis=0)
        out_ref[pl.ds(tb0, 8), H:2 * H] = jnp.concatenate(hb_rows[::-1], axis=0)
        return (h_f, c_f, h_b, c_b)

    z = jnp.zeros((1, H), jnp.float32)
    jax.lax.fori_loop(0, S // 8, step8, (z, z, z, z))


def _lstm_layer(x, pf, pb):
    din = x.shape[1]
    dpad = -(-din // 128) * 128
    if dpad != din:
        x = jnp.pad(x, ((0, 0), (0, dpad - din)))
    args = [x]
    for p in (pf, pb):
        wihT = jnp.transpose(p["Wih"])
        if dpad != din:
            wihT = jnp.pad(wihT, ((0, dpad - din), (0, 0)))
        args.append(wihT.astype(jnp.bfloat16))
        args.append(jnp.transpose(p["Whh"]).astype(jnp.bfloat16))
        args.append((p["bih"] + p["bhh"]).reshape(1, G))
    return pl.pallas_call(
        _lstm_layer_body,
        out_shape=jax.ShapeDtypeStruct((S, 2 * H), jnp.float32),
        scratch_shapes=[pltpu.VMEM((S, G), jnp.float32),
                        pltpu.VMEM((S, G), jnp.float32)],
    )(*args)


def _ab_body(emb_ref, embT_ref, w1aT, w1b, b1, a_out, bT_out):
    a_out[:] = jnp.dot(emb_ref[:].astype(jnp.bfloat16), w1aT[:],
                       preferred_element_type=jnp.float32) + b1[:]
    bT_out[:] = jnp.dot(w1b[:], embT_ref[:].astype(jnp.bfloat16),
                        preferred_element_type=jnp.float32)


_BI = 8


def _mlp_body(a_ref, bT_ref, w2, b2c, w3r, b3, sm_ref, logn_ref, nacc_ref):
    p = pl.program_id(0)
    np_ = pl.num_programs(0)
    jlane = jax.lax.broadcasted_iota(jnp.int32, (1, S), 1)

    @pl.when(p == 0)
    def _():
        nacc_ref[:] = jnp.zeros((1, S), jnp.float32)

    aT = a_ref[:].T
    bTv = bT_ref[:]
    h2s = []
    for k in range(_BI):
        acol = aT[:, k:k + 1]
        h1 = jnp.maximum(bTv + acol, 0.0)
        h2 = jnp.maximum(
            jnp.dot(w2[:], h1.astype(jnp.bfloat16),
                    preferred_element_type=jnp.float32) + b2c[:], 0.0)
        h2s.append(h2.astype(jnp.bfloat16))
    h2cat = jnp.concatenate(h2s, axis=1)
    rows = jnp.maximum(
        jnp.dot(w3r[:], h2cat, preferred_element_type=jnp.float32) + b3[:], 0.0)
    block = jnp.concatenate(
        [rows[:, k * S:(k + 1) * S] for k in range(_BI)], axis=0)
    ii = p * _BI + jax.lax.broadcasted_iota(jnp.int32, (_BI, S), 0)
    jj = jax.lax.broadcasted_iota(jnp.int32, (_BI, S), 1)
    offdiag = jj != ii
    blockm = jnp.where(offdiag & (jj >= 1), block, 0.0)
    sm_ref[:] = blockm
    nacc_ref[:] += jnp.sum(jnp.where(offdiag, jnp.exp(blockm), 0.0),
                           axis=0, keepdims=True)

    @pl.when(p == np_ - 1)
    def _():
        logn_ref[:] = jnp.log(nacc_ref[:])


def _pairwise_scores(emb, mlp_params):
    d = 2 * H
    w1 = mlp_params["W1"]
    a, bT = pl.pallas_call(
        _ab_body,
        out_shape=[jax.ShapeDtypeStruct((S, H), jnp.float32),
                   jax.ShapeDtypeStruct((H, S), jnp.float32)],
    )(emb, jnp.transpose(emb), jnp.transpose(w1[:, :d]).astype(jnp.bfloat16),
      w1[:, d:].astype(jnp.bfloat16), mlp_params["b1"].reshape(1, H))

    nprog = S // _BI
    sm, logn = pl.pallas_call(
        _mlp_body,
        grid=(nprog,),
        in_specs=[
            pl.BlockSpec((_BI, H), lambda p: (p, 0)),
            pl.BlockSpec((H, S), lambda p: (0, 0)),
            pl.BlockSpec((128, H), lambda p: (0, 0)),
            pl.BlockSpec((128, 1), lambda p: (0, 0)),
            pl.BlockSpec((1, 128), lambda p: (0, 0)),
            pl.BlockSpec((1, 1), lambda p: (0, 0)),
        ],
        out_specs=[
            pl.BlockSpec((_BI, S), lambda p: (p, 0)),
            pl.BlockSpec((1, S), lambda p: (0, 0)),
        ],
        out_shape=[jax.ShapeDtypeStruct((S, S), jnp.float32),
                   jax.ShapeDtypeStruct((1, S), jnp.float32)],
        scratch_shapes=[pltpu.VMEM((1, S), jnp.float32)],
    )(a, bT, mlp_params["W2"].astype(jnp.bfloat16),
      mlp_params["b2"].reshape(128, 1), mlp_params["W3"].astype(jnp.bfloat16),
      mlp_params["b3"].reshape(1, 1))
    return sm, logn[0, :]


def _loss_gather_sc(smflat, logn, idx1, idx2, w):
    info = plsc.get_sparse_core_info()
    nc, ns, nl = info.num_cores, info.num_subcores, info.num_lanes
    nw = nc * ns

    @functools.partial(
        pl.kernel,
        mesh=plsc.VectorSubcoreMesh(core_axis_name="c", subcore_axis_name="s"),
        out_type=jax.ShapeDtypeStruct((nw, nl), jnp.float32),
        scratch_types=[
            pltpu.VMEM((nl,), jnp.int32),
            pltpu.VMEM((nl,), jnp.int32),
            pltpu.VMEM((nl,), jnp.float32),
            pltpu.VMEM((nl,), jnp.float32),
            pltpu.VMEM((nl,), jnp.float32),
            pltpu.VMEM((nl,), jnp.float32),
            pltpu.SemaphoreType.DMA,
        ],
    )
    def body(smflat_hbm, logn_hbm, idx1_hbm, idx2_hbm, w_hbm, out_hbm,
             idx1_v, idx2_v, w_v, smv_v, lnv_v, res_v, sem):
        wid = lax.axis_index("s") * nc + lax.axis_index("c")
        base = wid * nl
        pltpu.sync_copy(idx1_hbm.at[pl.ds(base, nl)], idx1_v)
        pltpu.sync_copy(idx2_hbm.at[pl.ds(base, nl)], idx2_v)
        pltpu.sync_copy(w_hbm.at[pl.ds(base, nl)], w_v)
        pltpu.async_copy(smflat_hbm.at[idx1_v], smv_v, sem).wait()
        pltpu.async_copy(logn_hbm.at[idx2_v], lnv_v, sem).wait()
        res_v[...] = (lnv_v[...] - smv_v[...]) * w_v[...]
        pltpu.sync_copy(res_v, out_hbm.at[wid])

    return body(smflat, logn, idx1, idx2, w)


def kernel(sentence_embedding, real_dependency_tree, lstm_params, mlp_params):
    emb = _lstm_layer(sentence_embedding, lstm_params["l0_f"], lstm_params["l0_b"])
    emb = _lstm_layer(emb, lstm_params["l1_f"], lstm_params["l1_b"])
    sm, logn = _pairwise_scores(emb, mlp_params)

    tree = real_dependency_tree.astype(jnp.int32)
    v1p = jnp.concatenate([tree[1:, 0], jnp.zeros((1,), jnp.int32)])
    v2p = jnp.concatenate([tree[1:, 1], jnp.zeros((1,), jnp.int32)])
    w = jnp.where(jnp.arange(S) < S - 1, 1.0 / (S - 1), 0.0).astype(jnp.float32)
    terms = _loss_gather_sc(sm.reshape(S * S), logn, v1p * S + v2p, v2p, w)
    loss = jnp.sum(terms)
    return loss, sm
